# initial kernel scaffold (unmeasured)
import jax
import jax.numpy as jnp
from jax import lax
from jax.experimental import pallas as pl
from jax.experimental.pallas import tpu as pltpu


def kernel(
    t,
):
    def body(*refs):
        pass

    out_shape = jax.ShapeDtypeStruct(..., jnp.float32)
    return pl.pallas_call(body, out_shape=out_shape)(...)


# baseline (device time: 13789 ns/iter reference)
import jax
import jax.numpy as jnp
from jax import lax
from jax.experimental import pallas as pl
from jax.experimental.pallas import tpu as pltpu

N_DEV = 4


def kernel(t):
    m, n = t.shape

    def body(x_ref, out_ref, comm_ref, send_sems, recv_sems):
        p = lax.axis_index("i")
        partner_a = p ^ 1
        partner_b = 3 - p

        barrier_sem = pltpu.get_barrier_semaphore()
        for prt in [partner_a, partner_b]:
            pl.semaphore_signal(
                barrier_sem, inc=1,
                device_id=(prt,), device_id_type=pl.DeviceIdType.MESH,
            )
        pl.semaphore_wait(barrier_sem, 2)

        comm_ref[0, :, :] = x_ref[:, :].astype(jnp.bfloat16)
        rdma_a = pltpu.make_async_remote_copy(
            src_ref=comm_ref.at[0],
            dst_ref=comm_ref.at[1],
            send_sem=send_sems.at[0],
            recv_sem=recv_sems.at[0],
            device_id=(partner_a,),
            device_id_type=pl.DeviceIdType.MESH,
        )
        rdma_a.start()
        rdma_a.wait()
        comm_ref[2, :, :] = comm_ref[0, :, :] + comm_ref[1, :, :]

        rdma_b = pltpu.make_async_remote_copy(
            src_ref=comm_ref.at[2],
            dst_ref=comm_ref.at[3],
            send_sem=send_sems.at[1],
            recv_sem=recv_sems.at[1],
            device_id=(partner_b,),
            device_id_type=pl.DeviceIdType.MESH,
        )
        rdma_b.start()
        rdma_b.wait()

        s = comm_ref[2, :, :].astype(jnp.float32) + comm_ref[3, :, :].astype(
            jnp.float32
        )
        r = jnp.maximum(s, 0.0)
        out_ref[:, :] = jnp.tanh(s) * s * s + r * r * r


    return pl.pallas_call(
        body,
        out_shape=jax.ShapeDtypeStruct((m, n), jnp.float32),
        in_specs=[pl.BlockSpec(memory_space=pltpu.VMEM)],
        out_specs=pl.BlockSpec(memory_space=pltpu.VMEM),
        scratch_shapes=[
            pltpu.VMEM((4, m, n), jnp.bfloat16),
            pltpu.SemaphoreType.DMA((2,)),
            pltpu.SemaphoreType.DMA((2,)),
        ],
        compiler_params=pltpu.CompilerParams(collective_id=0),
    )(t)


# device time: 10996 ns/iter; 1.2540x vs baseline; 1.2540x over previous
import jax
import jax.numpy as jnp
from jax import lax
from jax.experimental import pallas as pl
from jax.experimental.pallas import tpu as pltpu

N_DEV = 4


def kernel(t):
    m, n = t.shape
    h = m // 2

    def body(x_ref, out_ref, comm_ref, send_sems, recv_sems):
        p = lax.axis_index("i")
        pa = p ^ 1
        pb = 3 - p

        barrier_sem = pltpu.get_barrier_semaphore()
        for prt in [pa, pb]:
            pl.semaphore_signal(
                barrier_sem, inc=1,
                device_id=(prt,), device_id_type=pl.DeviceIdType.MESH,
            )
        comm_ref[0, :, :] = x_ref[:h, :].astype(jnp.bfloat16)
        comm_ref[2, :, :] = x_ref[h:, :].astype(jnp.bfloat16)
        pl.semaphore_wait(barrier_sem, 2)

        r1a = pltpu.make_async_remote_copy(
            src_ref=comm_ref.at[0], dst_ref=comm_ref.at[1],
            send_sem=send_sems.at[0], recv_sem=recv_sems.at[0],
            device_id=(pa,), device_id_type=pl.DeviceIdType.MESH,
        )
        r1b = pltpu.make_async_remote_copy(
            src_ref=comm_ref.at[2], dst_ref=comm_ref.at[3],
            send_sem=send_sems.at[1], recv_sem=recv_sems.at[1],
            device_id=(pb,), device_id_type=pl.DeviceIdType.MESH,
        )
        r1a.start()
        r1b.start()

        r1a.wait_recv()
        comm_ref[4, :, :] = comm_ref[0, :, :] + comm_ref[1, :, :]
        r2b = pltpu.make_async_remote_copy(
            src_ref=comm_ref.at[4], dst_ref=comm_ref.at[5],
            send_sem=send_sems.at[2], recv_sem=recv_sems.at[2],
            device_id=(pb,), device_id_type=pl.DeviceIdType.MESH,
        )
        r2b.start()

        r1b.wait_recv()
        comm_ref[6, :, :] = comm_ref[2, :, :] + comm_ref[3, :, :]
        r2a = pltpu.make_async_remote_copy(
            src_ref=comm_ref.at[6], dst_ref=comm_ref.at[7],
            send_sem=send_sems.at[3], recv_sem=recv_sems.at[3],
            device_id=(pa,), device_id_type=pl.DeviceIdType.MESH,
        )
        r2a.start()

        r2b.wait_recv()
        s0 = comm_ref[4, :, :].astype(jnp.float32) + comm_ref[5, :, :].astype(
            jnp.float32
        )
        r0 = jnp.maximum(s0, 0.0)
        out_ref[:h, :] = jnp.tanh(s0) * s0 * s0 + r0 * r0 * r0

        r2a.wait_recv()
        s1 = comm_ref[6, :, :].astype(jnp.float32) + comm_ref[7, :, :].astype(
            jnp.float32
        )
        r1 = jnp.maximum(s1, 0.0)
        out_ref[h:, :] = jnp.tanh(s1) * s1 * s1 + r1 * r1 * r1

        r1a.wait_send()
        r1b.wait_send()
        r2b.wait_send()
        r2a.wait_send()

    return pl.pallas_call(
        body,
        out_shape=jax.ShapeDtypeStruct((m, n), jnp.float32),
        in_specs=[pl.BlockSpec(memory_space=pltpu.VMEM)],
        out_specs=pl.BlockSpec(memory_space=pltpu.VMEM),
        scratch_shapes=[
            pltpu.VMEM((8, h, n), jnp.bfloat16),
            pltpu.SemaphoreType.DMA((4,)),
            pltpu.SemaphoreType.DMA((4,)),
        ],
        compiler_params=pltpu.CompilerParams(collective_id=0),
    )(t)


# device time: 10276 ns/iter; 1.3419x vs baseline; 1.0701x over previous
import jax
import jax.numpy as jnp
from jax import lax
from jax.experimental import pallas as pl
from jax.experimental.pallas import tpu as pltpu

N_DEV = 4
N_Q = 4


def kernel(t):
    m, n = t.shape
    q = m // N_Q

    def body(x_ref, out_ref, comm_ref, send_sems, recv_sems):
        p = lax.axis_index("i")
        pa = p ^ 1
        pb = 3 - p
        part1 = [pa, pa, pb, pb]
        part2 = [pb, pb, pa, pa]

        def mk(src_slot, dst_slot, sem, dev):
            return pltpu.make_async_remote_copy(
                src_ref=comm_ref.at[src_slot],
                dst_ref=comm_ref.at[dst_slot],
                send_sem=send_sems.at[sem],
                recv_sem=recv_sems.at[sem],
                device_id=(dev,),
                device_id_type=pl.DeviceIdType.MESH,
            )

        r1 = [mk(i, 4 + i, i, part1[i]) for i in range(N_Q)]
        r2 = [mk(8 + i, 12 + i, 4 + i, part2[i]) for i in range(N_Q)]

        barrier_sem = pltpu.get_barrier_semaphore()
        for prt in [pa, pb]:
            pl.semaphore_signal(
                barrier_sem, inc=1,
                device_id=(prt,), device_id_type=pl.DeviceIdType.MESH,
            )
        comm_ref[0, :, :] = x_ref[0 * q : 1 * q, :].astype(jnp.bfloat16)
        comm_ref[2, :, :] = x_ref[2 * q : 3 * q, :].astype(jnp.bfloat16)
        pl.semaphore_wait(barrier_sem, 2)

        r1[0].start()
        r1[2].start()
        comm_ref[1, :, :] = x_ref[1 * q : 2 * q, :].astype(jnp.bfloat16)
        comm_ref[3, :, :] = x_ref[3 * q : 4 * q, :].astype(jnp.bfloat16)
        r1[1].start()
        r1[3].start()

        for i in [0, 2, 1, 3]:
            r1[i].wait_recv()
            comm_ref[8 + i, :, :] = comm_ref[i, :, :] + comm_ref[4 + i, :, :]
            r2[i].start()

        for i in [0, 2, 1, 3]:
            r2[i].wait_recv()
            s = comm_ref[8 + i, :, :].astype(jnp.float32) + comm_ref[
                12 + i, :, :
            ].astype(jnp.float32)
            r = jnp.maximum(s, 0.0)
            out_ref[i * q : (i + 1) * q, :] = jnp.tanh(s) * s * s + r * r * r

        for i in range(N_Q):
            r1[i].wait_send()
            r2[i].wait_send()

    return pl.pallas_call(
        body,
        out_shape=jax.ShapeDtypeStruct((m, n), jnp.float32),
        in_specs=[pl.BlockSpec(memory_space=pltpu.VMEM)],
        out_specs=pl.BlockSpec(memory_space=pltpu.VMEM),
        scratch_shapes=[
            pltpu.VMEM((16, q, n), jnp.bfloat16),
            pltpu.SemaphoreType.DMA((8,)),
            pltpu.SemaphoreType.DMA((8,)),
        ],
        compiler_params=pltpu.CompilerParams(collective_id=0),
    )(t)


# device time: 10016 ns/iter; 1.3767x vs baseline; 1.0260x over previous
import jax
import jax.numpy as jnp
from jax import lax
from jax.experimental import pallas as pl
from jax.experimental.pallas import tpu as pltpu

N_DEV = 4
N_C = 8


def kernel(t):
    m, n = t.shape
    c = m // N_C
    half = N_C // 2

    def body(x_ref, out_ref, comm_ref, send_sems, recv_sems):
        p = lax.axis_index("i")
        pa = p ^ 1
        pb = 3 - p
        part1 = [pa if i < half else pb for i in range(N_C)]
        part2 = [pb if i < half else pa for i in range(N_C)]
        order = [k + j * half for k in range(half) for j in (0, 1)]

        def mk(src_slot, dst_slot, sem, dev):
            return pltpu.make_async_remote_copy(
                src_ref=comm_ref.at[src_slot],
                dst_ref=comm_ref.at[dst_slot],
                send_sem=send_sems.at[sem],
                recv_sem=recv_sems.at[sem],
                device_id=(dev,),
                device_id_type=pl.DeviceIdType.MESH,
            )

        r1 = [mk(i, N_C + i, i, part1[i]) for i in range(N_C)]
        r2 = [mk(2 * N_C + i, 3 * N_C + i, N_C + i, part2[i]) for i in range(N_C)]

        barrier_sem = pltpu.get_barrier_semaphore()
        for prt in [pa, pb]:
            pl.semaphore_signal(
                barrier_sem, inc=1,
                device_id=(prt,), device_id_type=pl.DeviceIdType.MESH,
            )
        comm_ref[order[0], :, :] = x_ref[
            order[0] * c : (order[0] + 1) * c, :
        ].astype(jnp.bfloat16)
        comm_ref[order[1], :, :] = x_ref[
            order[1] * c : (order[1] + 1) * c, :
        ].astype(jnp.bfloat16)
        pl.semaphore_wait(barrier_sem, 2)

        r1[order[0]].start()
        r1[order[1]].start()
        for i in order[2:]:
            comm_ref[i, :, :] = x_ref[i * c : (i + 1) * c, :].astype(jnp.bfloat16)
            r1[i].start()

        for i in order:
            r1[i].wait_recv()
            comm_ref[2 * N_C + i, :, :] = (
                comm_ref[i, :, :] + comm_ref[N_C + i, :, :]
            )
            r2[i].start()

        for i in order:
            r2[i].wait_recv()
            s = (
                comm_ref[2 * N_C + i, :, :] + comm_ref[3 * N_C + i, :, :]
            ).astype(jnp.float32)
            r = jnp.maximum(s, 0.0)
            out_ref[i * c : (i + 1) * c, :] = jnp.tanh(s) * s * s + r * r * r

        for i in range(N_C):
            r1[i].wait_send()
            r2[i].wait_send()

    return pl.pallas_call(
        body,
        out_shape=jax.ShapeDtypeStruct((m, n), jnp.float32),
        in_specs=[pl.BlockSpec(memory_space=pltpu.VMEM)],
        out_specs=pl.BlockSpec(memory_space=pltpu.VMEM),
        scratch_shapes=[
            pltpu.VMEM((4 * N_C, c, n), jnp.bfloat16),
            pltpu.SemaphoreType.DMA((2 * N_C,)),
            pltpu.SemaphoreType.DMA((2 * N_C,)),
        ],
        compiler_params=pltpu.CompilerParams(collective_id=0),
    )(t)
